# Initial kernel scaffold; baseline (speedup 1.0000x reference)
import jax
import jax.numpy as jnp
from jax.experimental import pallas as pl


def kernel(indices, mask, emb, W1, b1, W2, b2, Wc, bc):
    raise NotImplementedError("write your pallas kernel here")


# speedup vs baseline: 2.1757x; 2.1757x over previous
import functools

import jax
import jax.numpy as jnp
from jax import lax
from jax.experimental import pallas as pl
from jax.experimental.pallas import tpu as pltpu
from jax.experimental.pallas import tpu_sc as plsc

_NC = 2
_NS = 16
_NW = _NC * _NS
_CH = 128


@functools.lru_cache(maxsize=None)
def _make_gather(V, D, N):
    per_w = N // _NW
    nch = per_w // _CH
    mesh = plsc.VectorSubcoreMesh(core_axis_name="c", subcore_axis_name="s")

    @functools.partial(
        pl.kernel,
        mesh=mesh,
        out_type=jax.ShapeDtypeStruct((N, D), jnp.float32),
        scratch_types=[
            pltpu.VMEM((nch, _CH), jnp.int32),
            pltpu.VMEM((_CH, D), jnp.float32),
            pltpu.SemaphoreType.DMA,
        ],
    )
    def gather_k(idx_hbm, table_hbm, out_hbm, idx_v, rows_v, sem):
        wid = lax.axis_index("s") * _NC + lax.axis_index("c")
        pltpu.sync_copy(idx_hbm.at[wid], idx_v)
        base = wid * per_w

        def body(i, carry):
            pltpu.async_copy(table_hbm.at[idx_v.at[i]], rows_v, sem).wait()
            pltpu.sync_copy(rows_v, out_hbm.at[pl.ds(base + i * _CH, _CH)])
            return carry

        lax.fori_loop(0, nch, body, 0)

    return gather_k


def _dense_body(L, BB, x_ref, m_ref, W1_ref, b1_ref, W2_ref, b2_ref,
                Wc_ref, bc_ref, o_ref):
    x = x_ref[...]
    h = jnp.dot(x, W1_ref[...], preferred_element_type=jnp.float32)
    h = h + b1_ref[...]
    h = h * jax.nn.sigmoid(h)
    g = jnp.dot(h, W2_ref[...], preferred_element_type=jnp.float32)
    g = g + b2_ref[...]
    g = g * jax.nn.sigmoid(g)
    m = m_ref[0]
    T = BB * L
    r = lax.broadcasted_iota(jnp.int32, (BB, T), 0)
    c = lax.broadcasted_iota(jnp.int32, (BB, T), 1)
    S = jnp.where(c // L == r, jnp.broadcast_to(m, (BB, T)), 0.0)
    denom = jnp.maximum(jnp.sum(S, axis=1, keepdims=True), 1.0)
    pooled = jnp.dot(S, g, preferred_element_type=jnp.float32) / denom
    out = jnp.dot(pooled, Wc_ref[...], preferred_element_type=jnp.float32)
    o_ref[...] = out + bc_ref[...]


def _dense_call(B, L, D, H, C, BB, interpret=False):
    T = BB * L
    grid = (B // BB,)
    return pl.pallas_call(
        functools.partial(_dense_body, L, BB),
        grid=grid,
        in_specs=[
            pl.BlockSpec((T, D), lambda i: (i, 0)),
            pl.BlockSpec((1, 1, T), lambda i: (i, 0, 0)),
            pl.BlockSpec((D, H), lambda i: (0, 0)),
            pl.BlockSpec((1, H), lambda i: (0, 0)),
            pl.BlockSpec((H, H), lambda i: (0, 0)),
            pl.BlockSpec((1, H), lambda i: (0, 0)),
            pl.BlockSpec((H, C), lambda i: (0, 0)),
            pl.BlockSpec((1, C), lambda i: (0, 0)),
        ],
        out_specs=pl.BlockSpec((BB, C), lambda i: (i, 0)),
        out_shape=jax.ShapeDtypeStruct((B, C), jnp.float32),
        compiler_params=pltpu.CompilerParams(
            dimension_semantics=("arbitrary",),
        ),
        interpret=interpret,
    )


def kernel(indices, mask, emb, W1, b1, W2, b2, Wc, bc):
    B, L = indices.shape
    V, D = emb.shape
    H = W1.shape[1]
    C = Wc.shape[1]
    N = B * L
    BB = 8

    idx3 = indices.astype(jnp.int32).reshape(_NW, N // (_NW * _CH), _CH)
    x = _make_gather(V, D, N)(idx3, emb)

    maskf = mask.astype(jnp.float32).reshape(B // BB, 1, BB * L)
    logits = _dense_call(B, L, D, H, C, BB)(
        x, maskf, W1, b1.reshape(1, H), W2, b2.reshape(1, H),
        Wc, bc.reshape(1, C))
    return logits
